# Initial kernel scaffold; baseline (speedup 1.0000x reference)
import jax
import jax.numpy as jnp
from jax.experimental import pallas as pl


def kernel(query, support, q_l1, q_dummy_l, q_deg_l, q_r1, q_dummy_r, q_deg_r, s_l1, s_dummy_l, s_deg_l, s_r1, s_dummy_r, s_deg_r, params):
    raise NotImplementedError("write your pallas kernel here")


# speedup vs baseline: 2.2611x; 2.2611x over previous
import functools

import jax
import jax.numpy as jnp
from jax import lax
from jax.experimental import pallas as pl
from jax.experimental.pallas import tpu as pltpu
from jax.experimental.pallas import tpu_sc as plsc

E = 64
DM = 128
B = 1024
K = 50
FEW = 5
PAD = 1000000
NJ = 2304
NR = NJ * K
NW = 32
RPW = NR // NW
SPW = NJ // NW


def _sc_gather(emb, gtbl, idx_rel, idx_ent, idx_self, idx_gate):
    mesh = plsc.VectorSubcoreMesh(core_axis_name="c", subcore_axis_name="s")

    @functools.partial(
        pl.kernel,
        mesh=mesh,
        compiler_params=pltpu.CompilerParams(use_tc_tiling_on_sc=False),
        out_type=[
            jax.ShapeDtypeStruct((NR, E), jnp.float32),
            jax.ShapeDtypeStruct((NR, E), jnp.float32),
            jax.ShapeDtypeStruct((NJ, E), jnp.float32),
            jax.ShapeDtypeStruct((NR, 16), jnp.float32),
        ],
        scratch_types=[
            pltpu.VMEM((RPW,), jnp.int32),
            pltpu.VMEM((600, E), jnp.float32),
            pltpu.VMEM((720, 16), jnp.float32),
            pltpu.SemaphoreType.DMA,
        ],
    )
    def k(emb_h, gtbl_h, ixr_h, ixe_h, ixs_h, ixg_h,
          gr_h, ge_h, gs_h, gg_h, idx_v, rows_v, grows_v, sem):
        w = lax.axis_index("s") * 2 + lax.axis_index("c")
        b = w * RPW
        pltpu.sync_copy(ixr_h.at[pl.ds(b, RPW)], idx_v)
        for c in range(6):
            pltpu.async_copy(
                emb_h.at[idx_v.at[pl.ds(c * 600, 600)]], rows_v, sem).wait()
            pltpu.sync_copy(rows_v, gr_h.at[pl.ds(b + c * 600, 600)])
        pltpu.sync_copy(ixe_h.at[pl.ds(b, RPW)], idx_v)
        for c in range(6):
            pltpu.async_copy(
                emb_h.at[idx_v.at[pl.ds(c * 600, 600)]], rows_v, sem).wait()
            pltpu.sync_copy(rows_v, ge_h.at[pl.ds(b + c * 600, 600)])
        bs = w * SPW
        pltpu.sync_copy(ixs_h.at[pl.ds(bs, SPW)], idx_v.at[pl.ds(0, SPW)])
        pltpu.async_copy(
            emb_h.at[idx_v.at[pl.ds(0, SPW)]], rows_v.at[pl.ds(0, SPW)],
            sem).wait()
        pltpu.sync_copy(rows_v.at[pl.ds(0, SPW)], gs_h.at[pl.ds(bs, SPW)])
        pltpu.sync_copy(ixg_h.at[pl.ds(b, RPW)], idx_v)
        for c in range(5):
            pltpu.async_copy(
                gtbl_h.at[idx_v.at[pl.ds(c * 720, 720)]], grows_v, sem).wait()
            pltpu.sync_copy(grows_v, gg_h.at[pl.ds(b + c * 720, 720)])

    return k(emb, gtbl, idx_rel, idx_ent, idx_self, idx_gate)


def _enc_body(rel_ref, ent_ref, self_ref, rels_ref, relsr_ref, g16_ref,
              deg_ref, wr_ref, we_ref, cst_ref, out_ref):
    rel = rel_ref[...]
    ent = ent_ref[...]
    cst = cst_ref[...]
    bv = cst[0:1, 0:E]
    inv_temp = cst[1, 0]
    proj = (jnp.dot(rel, wr_ref[...], preferred_element_type=jnp.float32)
            + jnp.dot(ent, we_ref[...], preferred_element_type=jnp.float32)
            + bv)
    proj = jnp.where(proj >= 0, proj, 0.01 * proj)
    proj = jnp.where(relsr_ref[...] == PAD, 0.0, proj)
    proj3 = proj.reshape(128, K, E)
    self_e = self_ref[...]
    self3 = lax.broadcast_in_dim(self_e, (128, K, E), (0, 2))
    num = jnp.sum(self3 * proj3, axis=2)
    ns = jnp.sqrt(jnp.sum(self_e * self_e, 1, keepdims=True) + 1e-8)
    nn = jnp.sqrt(jnp.sum(proj3 * proj3, 2) + 1e-8)
    cos = num / (ns * nn + 1e-8)
    iota = lax.broadcasted_iota(jnp.int32, (128, K), 1)
    masked = cos
    acc = jnp.zeros((128, E), jnp.float32)
    for _ in range(10):
        m = jnp.max(masked, 1, keepdims=True)
        sel = jnp.min(jnp.where(masked == m, iota, K), 1, keepdims=True)
        oh = iota == sel
        oh3 = lax.broadcast_in_dim(oh.astype(jnp.float32), (128, K, E),
                                   (0, 1))
        acc = acc + jnp.sum(oh3 * proj3, axis=1)
        masked = jnp.where(oh, -jnp.inf, masked)
    agg = acc * 0.1
    g16 = g16_ref[...]
    rels = rels_ref[...]
    rid = jnp.where(rels == PAD, 0, rels)
    lane3 = lax.broadcast_in_dim(rid & 15, (128, K, 16), (0, 1))
    l16 = lax.broadcasted_iota(jnp.int32, (128, K, 16), 2)
    gval = jnp.sum(jnp.where(l16 == lane3, g16, 0.0), axis=2)
    gmean = jnp.sum(gval, axis=1, keepdims=True) * (1.0 / K)
    gate = jax.nn.sigmoid(gmean * inv_temp)
    gate = jnp.where(deg_ref[...] > 0, gate, 1.0)
    out_ref[...] = jnp.tanh(self_e + gate * agg)


def _tc_encode(rel_rows, ent_rows, self_rows, rels, rels_row, g16, deg,
               wr, we, cst):
    grid = (NJ // 128,)
    return pl.pallas_call(
        _enc_body,
        grid=grid,
        in_specs=[
            pl.BlockSpec((6400, E), lambda i: (i, 0)),
            pl.BlockSpec((6400, E), lambda i: (i, 0)),
            pl.BlockSpec((128, E), lambda i: (i, 0)),
            pl.BlockSpec((128, K), lambda i: (i, 0)),
            pl.BlockSpec((6400, 1), lambda i: (i, 0)),
            pl.BlockSpec((128, K, 16), lambda i: (i, 0, 0)),
            pl.BlockSpec((128, 1), lambda i: (i, 0)),
            pl.BlockSpec((E, E), lambda i: (0, 0)),
            pl.BlockSpec((E, E), lambda i: (0, 0)),
            pl.BlockSpec((8, 128), lambda i: (0, 0)),
        ],
        out_specs=pl.BlockSpec((128, E), lambda i: (i, 0)),
        out_shape=jax.ShapeDtypeStruct((NJ, E), jnp.float32),
    )(rel_rows, ent_rows, self_rows, rels, rels_row, g16, deg, wr, we, cst)


def _head_body(qv_ref, sv_ref, w1_ref, b1_ref, w2_ref, b2_ref, gam_ref,
               bet_ref, wih_ref, whh_ref, bias_ref, out_ref):
    gam = gam_ref[...]
    bet = bet_ref[...]

    def se(x):
        h = jnp.maximum(
            jnp.dot(x, w1_ref[...], preferred_element_type=jnp.float32)
            + b1_ref[...], 0.0)
        y = (jnp.dot(h, w2_ref[...], preferred_element_type=jnp.float32)
             + b2_ref[...] + x)
        mu = jnp.mean(y, 1, keepdims=True)
        var = jnp.mean((y - mu) ** 2, 1, keepdims=True)
        return gam * (y - mu) / jnp.sqrt(var + 1e-5) + bet

    sv = se(sv_ref[...])
    smask = lax.broadcasted_iota(jnp.int32, (8, 1), 0) < FEW
    sg = jnp.sum(jnp.where(smask, sv, 0.0), 0, keepdims=True) * (1.0 / FEW)
    qe = se(qv_ref[...])
    wih = wih_ref[...]
    whh = whh_ref[...]
    bias = bias_ref[...]
    sgb = jnp.broadcast_to(sg, (B, DM))
    h_r = jnp.zeros((B, 2 * DM), jnp.float32)
    c = jnp.zeros((B, 2 * DM), jnp.float32)
    h = qe
    for _ in range(4):
        gates = (jnp.dot(qe, wih, preferred_element_type=jnp.float32)
                 + jnp.dot(h_r, whh, preferred_element_type=jnp.float32)
                 + bias)
        i_g = jax.nn.sigmoid(gates[:, 0:256])
        f_g = jax.nn.sigmoid(gates[:, 256:512])
        g_g = jnp.tanh(gates[:, 512:768])
        o_g = jax.nn.sigmoid(gates[:, 768:1024])
        c = f_g * c + i_g * g_g
        h = qe + (o_g * jnp.tanh(c))[:, 0:DM]
        h_r = jnp.concatenate([h, sgb], axis=1)
    num = jnp.sum(h * sgb, 1, keepdims=True)
    den = (jnp.sqrt(jnp.sum(h * h, 1, keepdims=True) + 1e-8)
           * jnp.sqrt(jnp.sum(sg * sg) + 1e-8))
    out_ref[...] = num / den


def _tc_head(qv, sv8, w1t, b1, w2t, b2, gam, bet, wiht, whht, bias):
    return pl.pallas_call(
        _head_body,
        out_shape=jax.ShapeDtypeStruct((B, 1), jnp.float32),
    )(qv, sv8, w1t, b1, w2t, b2, gam, bet, wiht, whht, bias)


def kernel(query, support, q_l1, q_dummy_l, q_deg_l, q_r1, q_dummy_r,
           q_deg_r, s_l1, s_dummy_l, s_deg_l, s_r1, s_dummy_r, s_deg_r,
           params):
    f32 = jnp.float32

    def padj(x, val=0):
        pad = [(0, NJ - x.shape[0])] + [(0, 0)] * (x.ndim - 1)
        return jnp.pad(x.astype(jnp.int32), pad, constant_values=val)

    self_ids = padj(jnp.concatenate(
        [query[:, 0], query[:, 1], support[:, 0], support[:, 1]]))
    rels = padj(jnp.concatenate(
        [q_l1[:, :, 0], q_r1[:, :, 0], s_l1[:, :, 0], s_r1[:, :, 0]]))
    ents = padj(jnp.concatenate(
        [q_l1[:, :, 1], q_r1[:, :, 1], s_l1[:, :, 1], s_r1[:, :, 1]]))
    degs = padj(jnp.concatenate(
        [q_deg_l, q_deg_r, s_deg_l, s_deg_r]))[:, None]
    rid = jnp.where(rels == PAD, 0, rels)

    emb = params['symbol_emb'].astype(f32)
    gtbl = params['gate_w'].astype(f32).reshape(62500, 16)
    g_rel, g_ent, g_self, g_gate = _sc_gather(
        emb, gtbl, rels.reshape(-1), ents.reshape(-1), self_ids,
        (rid >> 4).reshape(-1))

    wt = params['gcn_w_W'].astype(f32).T
    cst = jnp.zeros((8, 128), f32)
    cst = cst.at[0, 0:E].set(
        params['gcn_w_b'].astype(f32) + params['gcn_b'].astype(f32))
    cst = cst.at[1, 0].set(1.0 / params['gate_temp'].astype(f32))
    out_vec = _tc_encode(
        g_rel, g_ent, g_self, rels, rels.reshape(NR, 1),
        g_gate.reshape(NJ, K, 16), degs, wt[0:E], wt[E:2 * E], cst)

    query_vec = jnp.concatenate([out_vec[0:B], out_vec[B:2 * B]], axis=1)
    sv = jnp.concatenate(
        [out_vec[2 * B:2 * B + FEW], out_vec[2 * B + FEW:2 * B + 2 * FEW]],
        axis=1)
    sv8 = jnp.pad(sv, ((0, 3), (0, 0)))
    res = _tc_head(
        query_vec, sv8,
        params['se_W1'].astype(f32).T, params['se_b1'].astype(f32)[None, :],
        params['se_W2'].astype(f32).T, params['se_b2'].astype(f32)[None, :],
        params['se_gamma'].astype(f32)[None, :],
        params['se_beta'].astype(f32)[None, :],
        params['lstm_Wih'].astype(f32).T, params['lstm_Whh'].astype(f32).T,
        (params['lstm_bih'].astype(f32)
         + params['lstm_bhh'].astype(f32))[None, :])
    return res.reshape(B)
